# baseline (speedup 1.0000x reference)
import functools

import jax
import jax.numpy as jnp
from jax import lax
from jax.experimental import pallas as pl
from jax.experimental.pallas import tpu as pltpu
from jax.experimental.pallas import tpu_sc as plsc

B = 1024
T = 200
D = 64
BT = B * T
NC = 2
NS = 16
NW = NC * NS
B_PER_W = BT // NW
CHUNK = 64
N_CHUNKS = B_PER_W // CHUNK
NBUF = 4
LANES = 16
GROUPS = D // LANES
HT = T // 2
POS_STAGE = 128

_mesh = plsc.VectorSubcoreMesh(core_axis_name="c", subcore_axis_name="s")


@functools.partial(
    pl.kernel,
    mesh=_mesh,
    out_type=jax.ShapeDtypeStruct((BT // 2, 2 * D), jnp.float32),
    scratch_types=[
        pltpu.VMEM((B_PER_W,), jnp.int32),
        pltpu.VMEM((B_PER_W,), jnp.int32),
        pltpu.VMEM((NBUF, CHUNK, 2 * D), jnp.float32),
        pltpu.VMEM((NBUF, CHUNK // 2, 2 * D), jnp.float32),
        pltpu.VMEM((POS_STAGE, 2 * D), jnp.float32),
    ] + [pltpu.SemaphoreType.DMA] * (2 * NBUF),
    compiler_params=pltpu.CompilerParams(use_tc_tiling_on_sc=True),
)
def _embed_sc(x_hbm, tok_hbm, pos_hbm, out_hbm, idx_v, pidx_v, rows_v,
              obuf_v, pos_v, g0, g1, g2, g3, o0, o1, o2, o3):
    gsems = (g0, g1, g2, g3)
    osems = (o0, o1, o2, o3)
    cid = lax.axis_index("c")
    sid = lax.axis_index("s")
    wid = sid * NC + cid
    base = wid * B_PER_W
    base2 = base // 2

    pltpu.sync_copy(x_hbm.at[pl.ds(pl.multiple_of(base, 128), B_PER_W)], idx_v)
    pltpu.sync_copy(pos_hbm.at[pl.ds(0, POS_STAGE)], pos_v)

    @plsc.parallel_loop(0, B_PER_W // LANES, unroll=8)
    def _pidx(q):
        sl = pl.ds(q * LANES, LANES)
        pidx_v[sl] = lax.shift_right_logical(idx_v[sl], 1)

    def gather(ch, b):
        pltpu.async_copy(
            tok_hbm.at[pidx_v.at[pl.ds(pl.multiple_of(ch * CHUNK, 64), CHUNK)]],
            rows_v.at[b], gsems[b])

    def wait_gather(ch, b):
        pltpu.make_async_copy(
            tok_hbm.at[pidx_v.at[pl.ds(pl.multiple_of(ch * CHUNK, 64), CHUNK)]],
            rows_v.at[b], gsems[b]).wait()

    def put(ch, b):
        pltpu.async_copy(
            obuf_v.at[b],
            out_hbm.at[pl.ds(pl.multiple_of(base2 + ch * (CHUNK // 2), 32), CHUNK // 2)],
            osems[b])

    def wait_put(b):
        pltpu.make_async_copy(
            obuf_v.at[b], out_hbm.at[pl.ds(pl.multiple_of(base2, 32), CHUNK // 2)],
            osems[b]).wait()

    for b in range(NBUF - 1):
        gather(b, b)

    def block_body(blk, carry):
        c0 = blk * NBUF
        for b in range(NBUF):
            ch = c0 + b
            nxt = ch + NBUF - 1
            bn = (b + NBUF - 1) % NBUF

            @pl.when(nxt < N_CHUNKS)
            def _():
                gather(nxt, bn)

            wait_gather(ch, b)

            @pl.when(ch >= NBUF)
            def _():
                wait_put(b)

            p0h = lax.rem(ch * (CHUNK // 2), HT)

            @plsc.parallel_loop(0, CHUNK // LANES, unroll=1)
            def _combine(q):
                par16 = idx_v[pl.ds(ch * CHUNK + q * LANES, LANES)]
                for j in range(LANES):
                    off = (par16[j] & 1) * D
                    r2 = q * (LANES // 2) + j // 2
                    rp = j & 1
                    prow = lax.rem(p0h + r2, HT)
                    for g in range(GROUPS):
                        dsl = pl.ds(rp * D + g * LANES, LANES)
                        ssl = pl.ds(off + g * LANES, LANES)
                        obuf_v[b, r2, dsl] = (
                            rows_v[b, q * LANES + j, ssl]
                            + pos_v[prow, dsl])

            put(ch, b)
        return carry

    lax.fori_loop(0, N_CHUNKS // NBUF, block_body, 0)

    for b in range(NBUF):
        wait_put(b)


def kernel(x, token_emb, pos_emb):
    xflat = x.reshape(BT).astype(jnp.int32)
    tok2 = token_emb.reshape(-1, 2 * D)
    pos2 = pos_emb.reshape(-1, 2 * D)
    out = _embed_sc(xflat, tok2, pos2)
    return out.reshape(B, T, D)


# speedup vs baseline: 1.1334x; 1.1334x over previous
import functools

import jax
import jax.numpy as jnp
from jax import lax
from jax.experimental import pallas as pl
from jax.experimental.pallas import tpu as pltpu
from jax.experimental.pallas import tpu_sc as plsc

B = 1024
T = 200
D = 64
BT = B * T
NC = 2
NS = 16
NW = NC * NS
B_PER_W = BT // NW
CHUNK = 64
N_CHUNKS = B_PER_W // CHUNK
NBUF = 4
LANES = 16
GROUPS = D // LANES
POS_PAD = T + CHUNK

_mesh = plsc.VectorSubcoreMesh(core_axis_name="c", subcore_axis_name="s")


@functools.partial(
    pl.kernel,
    mesh=_mesh,
    out_type=jax.ShapeDtypeStruct((BT, D), jnp.float32),
    scratch_types=[
        pltpu.VMEM((N_CHUNKS, CHUNK), jnp.int32),
        pltpu.VMEM((NBUF, CHUNK, D), jnp.float32),
        pltpu.VMEM((POS_PAD, D), jnp.float32),
    ] + [pltpu.SemaphoreType.DMA] * (2 * NBUF),
    compiler_params=pltpu.CompilerParams(
        use_tc_tiling_on_sc=False, skip_device_barrier=True),
)
def _embed_sc(x_hbm, tok_hbm, pos_hbm, out_hbm, idx_v, rows_v, pos_v,
              g0, g1, g2, g3, o0, o1, o2, o3):
    gsems = (g0, g1, g2, g3)
    osems = (o0, o1, o2, o3)
    cid = lax.axis_index("c")
    sid = lax.axis_index("s")
    wid = sid * NC + cid
    base = wid * B_PER_W

    pltpu.sync_copy(x_hbm.at[wid], idx_v)
    pltpu.sync_copy(pos_hbm.at[pl.ds(0, T)], pos_v.at[pl.ds(0, T)])
    pltpu.sync_copy(pos_hbm.at[pl.ds(0, CHUNK)], pos_v.at[pl.ds(T, CHUNK)])

    def gather(ch, b):
        pltpu.async_copy(tok_hbm.at[idx_v.at[ch]], rows_v.at[b], gsems[b])

    def wait_gather(ch, b):
        pltpu.make_async_copy(
            tok_hbm.at[idx_v.at[ch]], rows_v.at[b], gsems[b]).wait()

    def put(ch, b):
        pltpu.async_copy(
            rows_v.at[b], out_hbm.at[pl.ds(base + ch * CHUNK, CHUNK)],
            osems[b])

    def wait_put(b):
        pltpu.make_async_copy(
            rows_v.at[b], out_hbm.at[pl.ds(base, CHUNK)], osems[b]).wait()

    for b in range(NBUF - 1):
        gather(b, b)

    def block_body(blk, carry):
        c0 = blk * NBUF
        for b in range(NBUF):
            ch = c0 + b
            nxt = ch + NBUF - 1
            bn = (b + NBUF - 1) % NBUF

            @pl.when(nxt < N_CHUNKS)
            def _():
                @pl.when(nxt >= NBUF)
                def _():
                    wait_put(bn)
                gather(nxt, bn)

            wait_gather(ch, b)
            p0 = lax.rem(ch * CHUNK, T)

            @plsc.parallel_loop(0, CHUNK, unroll=8)
            def _row(r):
                for g in range(GROUPS):
                    sl = pl.ds(g * LANES, LANES)
                    rows_v[b, r, sl] = rows_v[b, r, sl] + pos_v[p0 + r, sl]

            put(ch, b)
        return carry

    lax.fori_loop(0, N_CHUNKS // NBUF, block_body, 0)

    for b in range(NBUF):
        wait_put(b)


def kernel(x, token_emb, pos_emb):
    xw = x.reshape(NW, N_CHUNKS, CHUNK).astype(jnp.int32)
    out = _embed_sc(xw, token_emb, pos_emb)
    return out.reshape(B, T, D)
